# Initial kernel scaffold; baseline (speedup 1.0000x reference)
import jax
import jax.numpy as jnp
from jax.experimental import pallas as pl


def kernel(x, targets, embedding_table):
    raise NotImplementedError("write your pallas kernel here")


# speedup vs baseline: 1.9236x; 1.9236x over previous
import functools

import jax
import jax.numpy as jnp
from jax import lax
from jax.experimental import pallas as pl
from jax.experimental.pallas import tpu as pltpu
from jax.experimental.pallas import tpu_sc as plsc

VOCAB = 1000
B, T = 1024, 50
NTOK = B * T
NC, NS = 2, 16
NW = NC * NS
TOK_PER_W = NTOK // NW
CHUNK = 32
NCHUNK = TOK_PER_W // CHUNK
GROUPS = CHUNK // 16


VPAD = 1024


def _sc_body(table_hbm, xf_hbm, tf_hbm, lse_hbm, out_hbm, part_hbm,
             xid_v, tgt_v, lse_v, rows0, rows1, acc_v,
             gsem0, gsem1, csem0, csem1):
    rows = (rows0, rows1)
    gsem = (gsem0, gsem1)
    csem = (csem0, csem1)
    wid = lax.axis_index("s") * NC + lax.axis_index("c")
    base = wid * TOK_PER_W
    pltpu.sync_copy(xf_hbm.at[pl.ds(base, TOK_PER_W)], xid_v)
    pltpu.sync_copy(tf_hbm.at[pl.ds(base, TOK_PER_W)], tgt_v)
    pltpu.sync_copy(lse_hbm, lse_v)
    acc = jnp.zeros((16,), jnp.float32)
    gathers = [pltpu.async_copy(
        table_hbm.at[xid_v.at[pl.ds(0, CHUNK)]], rows[0], gsem[0]), None]
    copies = [None, None]
    for c in range(NCHUNK):
        b = c & 1
        nb = b ^ 1
        gathers[b].wait()
        if c + 1 < NCHUNK:
            if c >= 1:
                copies[nb].wait()
            gathers[nb] = pltpu.async_copy(
                table_hbm.at[xid_v.at[pl.ds((c + 1) * CHUNK, CHUNK)]],
                rows[nb], gsem[nb])
        copies[b] = pltpu.async_copy(
            rows[b], out_hbm.at[pl.ds(base + c * CHUNK, CHUNK)], csem[b])
        for g in range(GROUPS):
            off = c * CHUNK + g * 16
            tok16 = xid_v[pl.ds(off, 16)]
            t16 = tgt_v[pl.ds(off, 16)]
            lseg = plsc.load_gather(lse_v, [tok16])
            row16 = jnp.arange(16, dtype=jnp.int32) + (g * 16)
            tv = plsc.load_gather(rows[b], [row16, t16])
            acc = acc + (lseg - tv)
    copies[0].wait()
    copies[1].wait()
    acc_v[...] = acc
    pltpu.sync_copy(acc_v, part_hbm.at[wid])


_sc_gather_loss = functools.partial(
    pl.kernel,
    out_type=[
        jax.ShapeDtypeStruct((NTOK, 1024), jnp.float32),
        jax.ShapeDtypeStruct((NW, 16), jnp.float32),
    ],
    mesh=plsc.VectorSubcoreMesh(core_axis_name="c", subcore_axis_name="s"),
    compiler_params=pltpu.CompilerParams(needs_layout_passes=False),
    scratch_types=[
        pltpu.VMEM((TOK_PER_W,), jnp.int32),
        pltpu.VMEM((TOK_PER_W,), jnp.int32),
        pltpu.VMEM((VOCAB,), jnp.float32),
        pltpu.VMEM((CHUNK, VPAD), jnp.float32),
        pltpu.VMEM((CHUNK, VPAD), jnp.float32),
        pltpu.VMEM((16,), jnp.float32),
        pltpu.SemaphoreType.DMA,
        pltpu.SemaphoreType.DMA,
        pltpu.SemaphoreType.DMA,
        pltpu.SemaphoreType.DMA,
    ],
)(_sc_body)


def _lse_body(tab_ref, out_ref):
    t = tab_ref[...]
    m = jnp.max(t, axis=1, keepdims=True)
    s = jnp.sum(jnp.exp(t - m), axis=1, keepdims=True)
    out_ref[...] = m + jnp.log(s)


def _fin_body(p_ref, o_ref):
    o_ref[0, 0] = jnp.sum(p_ref[...]) * (1.0 / NTOK)


def kernel(x, targets, embedding_table):
    xf = x.reshape(-1).astype(jnp.int32)
    tf = targets.reshape(-1).astype(jnp.int32)
    lse = pl.pallas_call(
        _lse_body,
        out_shape=jax.ShapeDtypeStruct((VOCAB, 1), jnp.float32),
    )(embedding_table).reshape(-1)
    table_p = jnp.pad(embedding_table, ((0, 0), (0, VPAD - VOCAB)))
    logits_flat, part = _sc_gather_loss(table_p, xf, tf, lse)
    loss2d = pl.pallas_call(
        _fin_body,
        out_shape=jax.ShapeDtypeStruct((1, 1), jnp.float32),
        out_specs=pl.BlockSpec(memory_space=pltpu.SMEM),
    )(part)
    return logits_flat[:, :VOCAB].reshape(B, T, VOCAB), loss2d[0, 0]
